# baseline (speedup 1.0000x reference)
import functools

import numpy as np
import jax
import jax.numpy as jnp
from jax import lax
from jax.experimental import pallas as pl
from jax.experimental.pallas import tpu as pltpu
from jax.experimental.pallas import tpu_sc as plsc

H = 256
DEPTH = 12
T = 24
S = 2 ** DEPTH - 1
LEAVES = 2 ** (DEPTH - 1)
LATENT = 64
NW = 32
B_GATHER = T * LEAVES


XW_BASES = [0, 24, 72, 168, 360, 744, 1512, 3072, 6144, 12288, 24576]


def _brev(r, bits):
    out = np.zeros_like(r)
    for k in range(bits):
        out = (out << 1) | ((r >> k) & 1)
    return out


def _build_perms():
    int_perm = np.zeros(B_GATHER, np.int64)
    t = np.arange(T)
    for l in range(DEPTH - 1):
        r = np.arange(2 ** l)
        j = _brev(r, l)
        rows = XW_BASES[l] + r[:, None] * T + t[None, :]
        nodes = t[None, :] * S + (2 ** l - 1) + j[:, None]
        int_perm[rows.reshape(-1)] = nodes.reshape(-1)
    r = np.arange(LEAVES)
    j = _brev(r, DEPTH - 1)
    leaf_perm = np.zeros(B_GATHER, np.int64)
    rows = r[:, None] * T + t[None, :]
    nodes = t[None, :] * S + (LEAVES - 1) + j[:, None]
    leaf_perm[rows.reshape(-1)] = nodes.reshape(-1)
    return jnp.asarray(int_perm, jnp.int32), jnp.asarray(leaf_perm, jnp.int32)


_INT_PERM, _LEAF_PERM = _build_perms()


def _tables_body(leaf_t_ref, w_ih_ref, b_ihh_ref, hc_ref):
    gates = lax.dot_general(
        leaf_t_ref[...], w_ih_ref[...], (((1,), (1,)), ((), ())),
        preferred_element_type=jnp.float32) + b_ihh_ref[...]
    c = jax.nn.sigmoid(gates[:, 0:H]) * jnp.tanh(gates[:, 2 * H:3 * H])
    h = jax.nn.sigmoid(gates[:, 3 * H:4 * H]) * jnp.tanh(c)
    hc_ref[:, 0:H] = h.astype(jnp.bfloat16)
    hc_ref[:, H:2 * H] = c.astype(jnp.bfloat16)


def _tables_call(leaf_table, W_ih, b_ihh):
    V = leaf_table.shape[0]
    return pl.pallas_call(
        _tables_body,
        out_shape=jax.ShapeDtypeStruct((V, 2 * H), jnp.bfloat16),
    )(leaf_table, W_ih, b_ihh)


def _sc_gather(table, idx):
    V, D = table.shape
    B = idx.shape[0]
    b_per_w = B // NW
    C = {128: 128, 256: 128, 512: 96}[D]
    n_chunks = b_per_w // C
    assert n_chunks * C == b_per_w, (B, D, b_per_w, C)
    mesh = plsc.VectorSubcoreMesh(core_axis_name="c", subcore_axis_name="s")

    @functools.partial(
        pl.kernel, mesh=mesh,
        out_type=jax.ShapeDtypeStruct((B, D), table.dtype),
        scratch_types=[
            pltpu.VMEM((b_per_w,), jnp.int32),
            pltpu.VMEM((C, D), table.dtype),
            pltpu.VMEM((C, D), table.dtype),
            pltpu.SemaphoreType.DMA,
            pltpu.SemaphoreType.DMA,
        ],
    )
    def k(table_hbm, idx_hbm, out_hbm, idx_v, rows0, rows1, sem0, sem1):
        wid = lax.axis_index("s") * 2 + lax.axis_index("c")
        base = wid * b_per_w
        pltpu.sync_copy(idx_hbm.at[pl.ds(base, b_per_w)], idx_v)
        bufs = (rows0, rows1)
        sems = (sem0, sem1)
        copies = [None, None]
        for j in range(n_chunks):
            s = j & 1
            copies[s] = pltpu.async_copy(
                table_hbm.at[idx_v.at[pl.ds(j * C, C)]], bufs[s], sems[s])
            if j > 0:
                p = (j - 1) & 1
                copies[p].wait()
                pltpu.sync_copy(bufs[p], out_hbm.at[pl.ds(base + (j - 1) * C, C)])
        last = (n_chunks - 1) & 1
        copies[last].wait()
        pltpu.sync_copy(bufs[last],
                        out_hbm.at[pl.ds(base + (n_chunks - 1) * C, C)])

    return k(table, idx)


def _lo(x):
    return lax.bitcast_convert_type(x << 16, jnp.float32)


def _hi(x):
    return lax.bitcast_convert_type(x & jnp.int32(-65536), jnp.float32)


def _pack(h, c):
    hb = lax.bitcast_convert_type(
        h.astype(jnp.bfloat16).astype(jnp.float32), jnp.int32)
    cb = lax.bitcast_convert_type(
        c.astype(jnp.bfloat16).astype(jnp.float32), jnp.int32)
    return lax.shift_right_logical(hb, 16) | cb


def _cell(w0, w1, emb, u_iou, u_f, w_iou, w_f, b_iou, b_f):
    h0, c0 = _lo(w0), _hi(w0)
    h1, c1 = _lo(w1), _hi(w1)
    h0b = h0.astype(jnp.bfloat16)
    h1b = h1.astype(jnp.bfloat16)
    x = emb.astype(jnp.bfloat16)
    xwf = jnp.dot(x, w_f, preferred_element_type=jnp.float32) + b_f
    e = jnp.dot(x, w_iou, preferred_element_type=jnp.float32) + b_iou
    f0 = jax.nn.sigmoid(xwf + jnp.dot(h0b, u_f, preferred_element_type=jnp.float32))
    f1 = jax.nn.sigmoid(xwf + jnp.dot(h1b, u_f, preferred_element_type=jnp.float32))
    fc = f0 * c0 + f1 * c1
    m = jnp.dot(h0b + h1b, u_iou, preferred_element_type=jnp.float32)
    c_new = (jax.nn.sigmoid(e[:, 0:H] + m[:, 0:H])
             * jnp.tanh(e[:, 2 * H:3 * H] + m[:, 2 * H:3 * H]) + fc)
    h_new = (jax.nn.sigmoid(e[:, H:2 * H] + m[:, H:2 * H])
             * jnp.tanh(c_new))
    return h_new, c_new


def _level_body(hc0_ref, hc1_ref, emb_ref, u_iou_ref, u_f_ref, w_iou_ref,
                w_f_ref, b_iou_ref, b_f_ref, out_ref):
    h_new, c_new = _cell(hc0_ref[...], hc1_ref[...], emb_ref[...],
                         u_iou_ref[...], u_f_ref[...], w_iou_ref[...],
                         w_f_ref[...], b_iou_ref[...], b_f_ref[...])
    out_ref[...] = _pack(h_new, c_new)


EMBW = 128


def _level_call(hc_even, hc_odd, odd_off, emb, emb_block_off, cnt, consts):
    blk = min(cnt, 768)
    return pl.pallas_call(
        _level_body,
        grid=(cnt // blk,),
        in_specs=[
            pl.BlockSpec((blk, H), lambda i: (i, 0)),
            pl.BlockSpec((blk, H), lambda i: (i + odd_off, 0)),
            pl.BlockSpec((blk, EMBW), lambda i: (i + emb_block_off, 0)),
            pl.BlockSpec((H, 3 * H), lambda i: (0, 0)),
            pl.BlockSpec((H, H), lambda i: (0, 0)),
            pl.BlockSpec((EMBW, 3 * H), lambda i: (0, 0)),
            pl.BlockSpec((EMBW, H), lambda i: (0, 0)),
            pl.BlockSpec((1, 3 * H), lambda i: (0, 0)),
            pl.BlockSpec((1, H), lambda i: (0, 0)),
        ],
        out_specs=pl.BlockSpec((blk, H), lambda i: (i, 0)),
        out_shape=jax.ShapeDtypeStruct((cnt, H), jnp.int32),
    )(hc_even, hc_odd, emb, *consts)


def _top_body(hc7_ref, emb_ref, u_iou_ref, u_f_ref, w_iou_ref, w_f_ref,
              b_iou_ref, b_f_ref, wm_ref, bm_ref, wv_ref, bv_ref, eps_ref,
              z_ref, zm_ref, zlv_ref):
    consts = (u_iou_ref[...], u_f_ref[...], w_iou_ref[...], w_f_ref[...],
              b_iou_ref[...], b_f_ref[...])
    hc = hc7_ref[...]
    h_new = None
    for l in range(6, -1, -1):
        cnt = T * 2 ** l
        w0 = hc[0:cnt]
        w1 = hc[cnt:2 * cnt]
        emb = emb_ref[XW_BASES[l]:XW_BASES[l] + cnt]
        h_new, c_new = _cell(w0, w1, emb, *consts)
        if l > 0:
            hc = _pack(h_new, c_new)
    hroot = h_new
    zm = jnp.dot(hroot, wm_ref[...], preferred_element_type=jnp.float32) + bm_ref[...]
    zlv = jnp.dot(hroot, wv_ref[...], preferred_element_type=jnp.float32) + bv_ref[...]
    std = jnp.exp(0.5 * zlv)
    z_ref[...] = eps_ref[...] * std + zm
    zm_ref[...] = zm
    zlv_ref[...] = zlv


def _top_call(hc7, emb_small, consts, Wm, bm, Wv, bv, eps):
    o = jax.ShapeDtypeStruct((T, LATENT), jnp.float32)
    return pl.pallas_call(_top_body, out_shape=[o, o, o])(
        hc7, emb_small, *consts, Wm, bm, Wv, bv, eps)


def kernel(features, node_order_bottomup, adjacency_list, edge_order_bottomup,
           vocabs, tree_sizes, res_table, leaf_table, W_ih, b_ih, b_hh,
           W_iou, b_iou, U_iou, W_f, b_f, U_f, Wm, bm, Wv, bv):
    b_ihh = (b_ih + b_hh).reshape(1, 4 * H)
    hc_table = _tables_call(leaf_table, W_ih, b_ihh)

    hc_packed = lax.bitcast_convert_type(
        jnp.stack([hc_table[:, 0:H], hc_table[:, H:2 * H]], axis=-1), jnp.int32)
    ids_leaf = jnp.take(features, _LEAF_PERM).astype(jnp.int32)
    ids_int = jnp.take(features, _INT_PERM).astype(jnp.int32)
    hc = _sc_gather(hc_packed, ids_leaf)
    emb_int = _sc_gather(res_table, ids_int)

    consts = (U_iou.astype(jnp.bfloat16), U_f.astype(jnp.bfloat16),
              W_iou.astype(jnp.bfloat16), W_f.astype(jnp.bfloat16),
              b_iou.reshape(1, 3 * H), b_f.reshape(1, H))
    for l in (10, 9, 8, 7):
        cnt = T * 2 ** l
        blk = min(cnt, 768)
        hc = _level_call(hc, hc, cnt // blk, emb_int, XW_BASES[l] // blk,
                         cnt, consts)

    eps = jax.random.normal(jax.random.key(42), (T, LATENT), jnp.float32)
    emb_small = lax.slice_in_dim(emb_int, 0, XW_BASES[7], axis=0)
    z, zm, zlv = _top_call(hc, emb_small, consts,
                           Wm, bm.reshape(1, LATENT),
                           Wv, bv.reshape(1, LATENT), eps)
    return (z, zm, zlv)


# speedup vs baseline: 1.0008x; 1.0008x over previous
import functools

import numpy as np
import jax
import jax.numpy as jnp
from jax import lax
from jax.experimental import pallas as pl
from jax.experimental.pallas import tpu as pltpu
from jax.experimental.pallas import tpu_sc as plsc

H = 256
DEPTH = 12
T = 24
S = 2 ** DEPTH - 1
LEAVES = 2 ** (DEPTH - 1)
LATENT = 64
NW = 32
B_GATHER = T * LEAVES


XW_BASES = [0, 24, 72, 168, 360, 744, 1512, 3072, 6144, 12288, 24576]


def _brev(r, bits):
    out = np.zeros_like(r)
    for k in range(bits):
        out = (out << 1) | ((r >> k) & 1)
    return out


def _build_perms():
    int_perm = np.zeros(B_GATHER, np.int64)
    t = np.arange(T)
    for l in range(DEPTH - 1):
        r = np.arange(2 ** l)
        j = _brev(r, l)
        rows = XW_BASES[l] + r[:, None] * T + t[None, :]
        nodes = t[None, :] * S + (2 ** l - 1) + j[:, None]
        int_perm[rows.reshape(-1)] = nodes.reshape(-1)
    r = np.arange(LEAVES)
    j = _brev(r, DEPTH - 1)
    leaf_perm = np.zeros(B_GATHER, np.int64)
    rows = r[:, None] * T + t[None, :]
    nodes = t[None, :] * S + (LEAVES - 1) + j[:, None]
    leaf_perm[rows.reshape(-1)] = nodes.reshape(-1)
    return jnp.asarray(int_perm, jnp.int32), jnp.asarray(leaf_perm, jnp.int32)


_INT_PERM, _LEAF_PERM = _build_perms()


def _tables_body(leaf_t_ref, w_ih_ref, b_ihh_ref, hc_ref):
    gates = lax.dot_general(
        leaf_t_ref[...], w_ih_ref[...], (((1,), (1,)), ((), ())),
        preferred_element_type=jnp.float32) + b_ihh_ref[...]
    c = jax.nn.sigmoid(gates[:, 0:H]) * jnp.tanh(gates[:, 2 * H:3 * H])
    h = jax.nn.sigmoid(gates[:, 3 * H:4 * H]) * jnp.tanh(c)
    hc_ref[:, 0:H] = h.astype(jnp.bfloat16)
    hc_ref[:, H:2 * H] = c.astype(jnp.bfloat16)


def _tables_call(leaf_table, W_ih, b_ihh):
    V = leaf_table.shape[0]
    return pl.pallas_call(
        _tables_body,
        out_shape=jax.ShapeDtypeStruct((V, 2 * H), jnp.bfloat16),
    )(leaf_table, W_ih, b_ihh)


def _sc_gather(table, idx):
    V, D = table.shape
    B = idx.shape[0]
    b_per_w = B // NW
    C = {128: 128, 256: 128, 512: 96}[D]
    n_chunks = b_per_w // C
    assert n_chunks * C == b_per_w, (B, D, b_per_w, C)
    mesh = plsc.VectorSubcoreMesh(core_axis_name="c", subcore_axis_name="s")

    @functools.partial(
        pl.kernel, mesh=mesh,
        out_type=jax.ShapeDtypeStruct((B, D), table.dtype),
        scratch_types=[
            pltpu.VMEM((b_per_w,), jnp.int32),
            pltpu.VMEM((C, D), table.dtype),
            pltpu.VMEM((C, D), table.dtype),
            pltpu.SemaphoreType.DMA,
            pltpu.SemaphoreType.DMA,
        ],
    )
    def k(table_hbm, idx_hbm, out_hbm, idx_v, rows0, rows1, sem0, sem1):
        wid = lax.axis_index("s") * 2 + lax.axis_index("c")
        base = wid * b_per_w
        pltpu.sync_copy(idx_hbm.at[pl.ds(base, b_per_w)], idx_v)
        bufs = (rows0, rows1)
        sems = (sem0, sem1)
        copies = [None, None]
        for j in range(n_chunks):
            s = j & 1
            copies[s] = pltpu.async_copy(
                table_hbm.at[idx_v.at[pl.ds(j * C, C)]], bufs[s], sems[s])
            if j > 0:
                p = (j - 1) & 1
                copies[p].wait()
                pltpu.sync_copy(bufs[p], out_hbm.at[pl.ds(base + (j - 1) * C, C)])
        last = (n_chunks - 1) & 1
        copies[last].wait()
        pltpu.sync_copy(bufs[last],
                        out_hbm.at[pl.ds(base + (n_chunks - 1) * C, C)])

    return k(table, idx)


def _sc_gather2(hc_table, emb_table, idx_leaf, idx_int):
    B = idx_leaf.shape[0]
    b_per_w = B // NW
    C = 128
    n_chunks = b_per_w // C
    DH = hc_table.shape[1]
    DE = emb_table.shape[1]
    mesh = plsc.VectorSubcoreMesh(core_axis_name="c", subcore_axis_name="s")

    @functools.partial(
        pl.kernel, mesh=mesh,
        out_type=[jax.ShapeDtypeStruct((B, DH), jnp.int32),
                  jax.ShapeDtypeStruct((B, DE), jnp.float32)],
        scratch_types=[
            pltpu.VMEM((b_per_w,), jnp.int32),
            pltpu.VMEM((b_per_w,), jnp.int32),
            pltpu.VMEM((C, DH), jnp.int32),
            pltpu.VMEM((C, DH), jnp.int32),
            pltpu.VMEM((C, DE), jnp.float32),
            pltpu.VMEM((C, DE), jnp.float32),
            pltpu.SemaphoreType.DMA,
            pltpu.SemaphoreType.DMA,
        ],
    )
    def k(hc_hbm, emb_hbm, idl_hbm, idi_hbm, hc_out, emb_out,
          idl_v, idi_v, h0, h1, e0, e1, sem0, sem1):
        wid = lax.axis_index("s") * 2 + lax.axis_index("c")
        base = wid * b_per_w
        pltpu.sync_copy(idl_hbm.at[pl.ds(base, b_per_w)], idl_v)
        pltpu.sync_copy(idi_hbm.at[pl.ds(base, b_per_w)], idi_v)
        for table, idx_v, bufs, out in (
                (hc_hbm, idl_v, (h0, h1), hc_out),
                (emb_hbm, idi_v, (e0, e1), emb_out)):
            sems = (sem0, sem1)
            copies = [None, None]
            for j in range(n_chunks):
                s = j & 1
                copies[s] = pltpu.async_copy(
                    table.at[idx_v.at[pl.ds(j * C, C)]], bufs[s], sems[s])
                if j > 0:
                    p = (j - 1) & 1
                    copies[p].wait()
                    pltpu.sync_copy(bufs[p],
                                    out.at[pl.ds(base + (j - 1) * C, C)])
            last = (n_chunks - 1) & 1
            copies[last].wait()
            pltpu.sync_copy(bufs[last],
                            out.at[pl.ds(base + (n_chunks - 1) * C, C)])

    return k(hc_table, emb_table, idx_leaf, idx_int)


def _lo(x):
    return lax.bitcast_convert_type(x << 16, jnp.float32)


def _hi(x):
    return lax.bitcast_convert_type(x & jnp.int32(-65536), jnp.float32)


def _pack(h, c):
    hb = lax.bitcast_convert_type(
        h.astype(jnp.bfloat16).astype(jnp.float32), jnp.int32)
    cb = lax.bitcast_convert_type(
        c.astype(jnp.bfloat16).astype(jnp.float32), jnp.int32)
    return lax.shift_right_logical(hb, 16) | cb


def _cell(w0, w1, emb, u_iou, u_f, w_iou, w_f, b_iou, b_f):
    h0, c0 = _lo(w0), _hi(w0)
    h1, c1 = _lo(w1), _hi(w1)
    h0b = h0.astype(jnp.bfloat16)
    h1b = h1.astype(jnp.bfloat16)
    x = emb.astype(jnp.bfloat16)
    xwf = jnp.dot(x, w_f, preferred_element_type=jnp.float32) + b_f
    e = jnp.dot(x, w_iou, preferred_element_type=jnp.float32) + b_iou
    f0 = jax.nn.sigmoid(xwf + jnp.dot(h0b, u_f, preferred_element_type=jnp.float32))
    f1 = jax.nn.sigmoid(xwf + jnp.dot(h1b, u_f, preferred_element_type=jnp.float32))
    fc = f0 * c0 + f1 * c1
    m = jnp.dot(h0b + h1b, u_iou, preferred_element_type=jnp.float32)
    c_new = (jax.nn.sigmoid(e[:, 0:H] + m[:, 0:H])
             * jnp.tanh(e[:, 2 * H:3 * H] + m[:, 2 * H:3 * H]) + fc)
    h_new = (jax.nn.sigmoid(e[:, H:2 * H] + m[:, H:2 * H])
             * jnp.tanh(c_new))
    return h_new, c_new


def _level_body(hc0_ref, hc1_ref, emb_ref, u_iou_ref, u_f_ref, w_iou_ref,
                w_f_ref, b_iou_ref, b_f_ref, out_ref):
    h_new, c_new = _cell(hc0_ref[...], hc1_ref[...], emb_ref[...],
                         u_iou_ref[...], u_f_ref[...], w_iou_ref[...],
                         w_f_ref[...], b_iou_ref[...], b_f_ref[...])
    out_ref[...] = _pack(h_new, c_new)


EMBW = 128


def _level_call(hc_even, hc_odd, odd_off, emb, emb_block_off, cnt, consts):
    blk = min(cnt, 1536)
    return pl.pallas_call(
        _level_body,
        grid=(cnt // blk,),
        in_specs=[
            pl.BlockSpec((blk, H), lambda i: (i, 0)),
            pl.BlockSpec((blk, H), lambda i: (i + odd_off, 0)),
            pl.BlockSpec((blk, EMBW), lambda i: (i + emb_block_off, 0)),
            pl.BlockSpec((H, 3 * H), lambda i: (0, 0)),
            pl.BlockSpec((H, H), lambda i: (0, 0)),
            pl.BlockSpec((EMBW, 3 * H), lambda i: (0, 0)),
            pl.BlockSpec((EMBW, H), lambda i: (0, 0)),
            pl.BlockSpec((1, 3 * H), lambda i: (0, 0)),
            pl.BlockSpec((1, H), lambda i: (0, 0)),
        ],
        out_specs=pl.BlockSpec((blk, H), lambda i: (i, 0)),
        out_shape=jax.ShapeDtypeStruct((cnt, H), jnp.int32),
    )(hc_even, hc_odd, emb, *consts)


def _top_body(hc7_ref, emb_ref, u_iou_ref, u_f_ref, w_iou_ref, w_f_ref,
              b_iou_ref, b_f_ref, wm_ref, bm_ref, wv_ref, bv_ref, eps_ref,
              z_ref, zm_ref, zlv_ref):
    consts = (u_iou_ref[...], u_f_ref[...], w_iou_ref[...], w_f_ref[...],
              b_iou_ref[...], b_f_ref[...])
    hc = hc7_ref[...]
    h_new = None
    for l in range(6, -1, -1):
        cnt = T * 2 ** l
        w0 = hc[0:cnt]
        w1 = hc[cnt:2 * cnt]
        emb = emb_ref[XW_BASES[l]:XW_BASES[l] + cnt]
        h_new, c_new = _cell(w0, w1, emb, *consts)
        if l > 0:
            hc = _pack(h_new, c_new)
    hroot = h_new
    zm = jnp.dot(hroot, wm_ref[...], preferred_element_type=jnp.float32) + bm_ref[...]
    zlv = jnp.dot(hroot, wv_ref[...], preferred_element_type=jnp.float32) + bv_ref[...]
    std = jnp.exp(0.5 * zlv)
    z_ref[...] = eps_ref[...] * std + zm
    zm_ref[...] = zm
    zlv_ref[...] = zlv


def _top_call(hc7, emb_small, consts, Wm, bm, Wv, bv, eps):
    o = jax.ShapeDtypeStruct((T, LATENT), jnp.float32)
    return pl.pallas_call(_top_body, out_shape=[o, o, o])(
        hc7, emb_small, *consts, Wm, bm, Wv, bv, eps)


def kernel(features, node_order_bottomup, adjacency_list, edge_order_bottomup,
           vocabs, tree_sizes, res_table, leaf_table, W_ih, b_ih, b_hh,
           W_iou, b_iou, U_iou, W_f, b_f, U_f, Wm, bm, Wv, bv):
    b_ihh = (b_ih + b_hh).reshape(1, 4 * H)
    hc_table = _tables_call(leaf_table, W_ih, b_ihh)

    hc_packed = lax.bitcast_convert_type(
        jnp.stack([hc_table[:, 0:H], hc_table[:, H:2 * H]], axis=-1), jnp.int32)
    ids_leaf = jnp.take(features, _LEAF_PERM).astype(jnp.int32)
    ids_int = jnp.take(features, _INT_PERM).astype(jnp.int32)
    hc, emb_int = _sc_gather2(hc_packed, res_table, ids_leaf, ids_int)

    consts = (U_iou.astype(jnp.bfloat16), U_f.astype(jnp.bfloat16),
              W_iou.astype(jnp.bfloat16), W_f.astype(jnp.bfloat16),
              b_iou.reshape(1, 3 * H), b_f.reshape(1, H))
    for l in (10, 9, 8, 7):
        cnt = T * 2 ** l
        blk = min(cnt, 1536)
        hc = _level_call(hc, hc, cnt // blk, emb_int, XW_BASES[l] // blk,
                         cnt, consts)

    eps = jax.random.normal(jax.random.key(42), (T, LATENT), jnp.float32)
    emb_small = lax.slice_in_dim(emb_int, 0, XW_BASES[7], axis=0)
    z, zm, zlv = _top_call(hc, emb_small, consts,
                           Wm, bm.reshape(1, LATENT),
                           Wv, bv.reshape(1, LATENT), eps)
    return (z, zm, zlv)


# speedup vs baseline: 1.0200x; 1.0192x over previous
import functools

import numpy as np
import jax
import jax.numpy as jnp
from jax import lax
from jax.experimental import pallas as pl
from jax.experimental.pallas import tpu as pltpu
from jax.experimental.pallas import tpu_sc as plsc

H = 256
DEPTH = 12
T = 24
S = 2 ** DEPTH - 1
LEAVES = 2 ** (DEPTH - 1)
LATENT = 64
NW = 32
B_GATHER = T * LEAVES


XW_BASES = [0, 24, 72, 168, 360, 744, 1512, 3072, 6144, 12288, 24576]


def _brev(r, bits):
    out = np.zeros_like(r)
    for k in range(bits):
        out = (out << 1) | ((r >> k) & 1)
    return out


def _build_perms():
    int_perm = np.zeros(B_GATHER, np.int64)
    t = np.arange(T)
    for l in range(DEPTH - 1):
        r = np.arange(2 ** l)
        j = _brev(r, l)
        rows = XW_BASES[l] + r[:, None] * T + t[None, :]
        nodes = t[None, :] * S + (2 ** l - 1) + j[:, None]
        int_perm[rows.reshape(-1)] = nodes.reshape(-1)
    r = np.arange(LEAVES)
    j = _brev(r, DEPTH - 1)
    leaf_perm = np.zeros(B_GATHER, np.int64)
    rows = r[:, None] * T + t[None, :]
    nodes = t[None, :] * S + (LEAVES - 1) + j[:, None]
    leaf_perm[rows.reshape(-1)] = nodes.reshape(-1)
    return jnp.asarray(int_perm, jnp.int32), jnp.asarray(leaf_perm, jnp.int32)


_INT_PERM, _LEAF_PERM = _build_perms()


def _tables_body(leaf_t_ref, w_ih_ref, b_ihh_ref, hc_ref):
    gates = lax.dot_general(
        leaf_t_ref[...], w_ih_ref[...], (((1,), (1,)), ((), ())),
        preferred_element_type=jnp.float32) + b_ihh_ref[...]
    c = jax.nn.sigmoid(gates[:, 0:H]) * jnp.tanh(gates[:, 2 * H:3 * H])
    h = jax.nn.sigmoid(gates[:, 3 * H:4 * H]) * jnp.tanh(c)
    hc_ref[:, 0:H] = h.astype(jnp.bfloat16)
    hc_ref[:, H:2 * H] = c.astype(jnp.bfloat16)


def _tables_call(leaf_table, W_ih, b_ihh):
    V = leaf_table.shape[0]
    return pl.pallas_call(
        _tables_body,
        out_shape=jax.ShapeDtypeStruct((V, 2 * H), jnp.bfloat16),
    )(leaf_table, W_ih, b_ihh)


def _sc_gather(table, idx):
    V, D = table.shape
    B = idx.shape[0]
    b_per_w = B // NW
    C = {128: 128, 256: 128, 512: 96}[D]
    n_chunks = b_per_w // C
    assert n_chunks * C == b_per_w, (B, D, b_per_w, C)
    mesh = plsc.VectorSubcoreMesh(core_axis_name="c", subcore_axis_name="s")

    @functools.partial(
        pl.kernel, mesh=mesh,
        out_type=jax.ShapeDtypeStruct((B, D), table.dtype),
        scratch_types=[
            pltpu.VMEM((b_per_w,), jnp.int32),
            pltpu.VMEM((C, D), table.dtype),
            pltpu.VMEM((C, D), table.dtype),
            pltpu.SemaphoreType.DMA,
            pltpu.SemaphoreType.DMA,
        ],
    )
    def k(table_hbm, idx_hbm, out_hbm, idx_v, rows0, rows1, sem0, sem1):
        wid = lax.axis_index("s") * 2 + lax.axis_index("c")
        base = wid * b_per_w
        pltpu.sync_copy(idx_hbm.at[pl.ds(base, b_per_w)], idx_v)
        bufs = (rows0, rows1)
        sems = (sem0, sem1)
        copies = [None, None]
        for j in range(n_chunks):
            s = j & 1
            copies[s] = pltpu.async_copy(
                table_hbm.at[idx_v.at[pl.ds(j * C, C)]], bufs[s], sems[s])
            if j > 0:
                p = (j - 1) & 1
                copies[p].wait()
                pltpu.sync_copy(bufs[p], out_hbm.at[pl.ds(base + (j - 1) * C, C)])
        last = (n_chunks - 1) & 1
        copies[last].wait()
        pltpu.sync_copy(bufs[last],
                        out_hbm.at[pl.ds(base + (n_chunks - 1) * C, C)])

    return k(table, idx)


def _lo(x):
    return lax.bitcast_convert_type(x << 16, jnp.float32)


def _hi(x):
    return lax.bitcast_convert_type(x & jnp.int32(-65536), jnp.float32)


def _pack(h, c):
    hb = lax.bitcast_convert_type(
        h.astype(jnp.bfloat16).astype(jnp.float32), jnp.int32)
    cb = lax.bitcast_convert_type(
        c.astype(jnp.bfloat16).astype(jnp.float32), jnp.int32)
    return lax.shift_right_logical(hb, 16) | cb


def _cell(w0, w1, emb, u_iou, u_f, w_iou, w_f, b_iou, b_f):
    h0, c0 = _lo(w0), _hi(w0)
    h1, c1 = _lo(w1), _hi(w1)
    h0b = h0.astype(jnp.bfloat16)
    h1b = h1.astype(jnp.bfloat16)
    x = emb.astype(jnp.bfloat16)
    xwf = jnp.dot(x, w_f, preferred_element_type=jnp.float32) + b_f
    e = jnp.dot(x, w_iou, preferred_element_type=jnp.float32) + b_iou
    f0 = jax.nn.sigmoid(xwf + jnp.dot(h0b, u_f, preferred_element_type=jnp.float32))
    f1 = jax.nn.sigmoid(xwf + jnp.dot(h1b, u_f, preferred_element_type=jnp.float32))
    fc = f0 * c0 + f1 * c1
    m = jnp.dot(h0b + h1b, u_iou, preferred_element_type=jnp.float32)
    c_new = (jax.nn.sigmoid(e[:, 0:H] + m[:, 0:H])
             * jnp.tanh(e[:, 2 * H:3 * H] + m[:, 2 * H:3 * H]) + fc)
    h_new = (jax.nn.sigmoid(e[:, H:2 * H] + m[:, H:2 * H])
             * jnp.tanh(c_new))
    return h_new, c_new


def _level_body(hc0_ref, hc1_ref, emb_ref, u_iou_ref, u_f_ref, w_iou_ref,
                w_f_ref, b_iou_ref, b_f_ref, out_ref):
    h_new, c_new = _cell(hc0_ref[...], hc1_ref[...], emb_ref[...],
                         u_iou_ref[...], u_f_ref[...], w_iou_ref[...],
                         w_f_ref[...], b_iou_ref[...], b_f_ref[...])
    out_ref[...] = _pack(h_new, c_new)


EMBW = 128


def _level_call(hc_even, hc_odd, odd_off, emb, emb_block_off, cnt, consts):
    blk = min(cnt, 1536)
    return pl.pallas_call(
        _level_body,
        grid=(cnt // blk,),
        in_specs=[
            pl.BlockSpec((blk, H), lambda i: (i, 0)),
            pl.BlockSpec((blk, H), lambda i: (i + odd_off, 0)),
            pl.BlockSpec((blk, EMBW), lambda i: (i + emb_block_off, 0)),
            pl.BlockSpec((H, 3 * H), lambda i: (0, 0)),
            pl.BlockSpec((H, H), lambda i: (0, 0)),
            pl.BlockSpec((EMBW, 3 * H), lambda i: (0, 0)),
            pl.BlockSpec((EMBW, H), lambda i: (0, 0)),
            pl.BlockSpec((1, 3 * H), lambda i: (0, 0)),
            pl.BlockSpec((1, H), lambda i: (0, 0)),
        ],
        out_specs=pl.BlockSpec((blk, H), lambda i: (i, 0)),
        out_shape=jax.ShapeDtypeStruct((cnt, H), jnp.int32),
    )(hc_even, hc_odd, emb, *consts)


def _top_body(hc7_ref, emb_ref, u_iou_ref, u_f_ref, w_iou_ref, w_f_ref,
              b_iou_ref, b_f_ref, wm_ref, bm_ref, wv_ref, bv_ref, eps_ref,
              z_ref, zm_ref, zlv_ref):
    consts = (u_iou_ref[...], u_f_ref[...], w_iou_ref[...], w_f_ref[...],
              b_iou_ref[...], b_f_ref[...])
    hc = hc7_ref[...]
    h_new = None
    for l in range(6, -1, -1):
        cnt = T * 2 ** l
        w0 = hc[0:cnt]
        w1 = hc[cnt:2 * cnt]
        emb = emb_ref[XW_BASES[l]:XW_BASES[l] + cnt]
        h_new, c_new = _cell(w0, w1, emb, *consts)
        if l > 0:
            hc = _pack(h_new, c_new)
    hroot = h_new
    zm = jnp.dot(hroot, wm_ref[...], preferred_element_type=jnp.float32) + bm_ref[...]
    zlv = jnp.dot(hroot, wv_ref[...], preferred_element_type=jnp.float32) + bv_ref[...]
    std = jnp.exp(0.5 * zlv)
    z_ref[...] = eps_ref[...] * std + zm
    zm_ref[...] = zm
    zlv_ref[...] = zlv


def _top_call(hc7, emb_small, consts, Wm, bm, Wv, bv, eps):
    o = jax.ShapeDtypeStruct((T, LATENT), jnp.float32)
    return pl.pallas_call(_top_body, out_shape=[o, o, o])(
        hc7, emb_small, *consts, Wm, bm, Wv, bv, eps)


def kernel(features, node_order_bottomup, adjacency_list, edge_order_bottomup,
           vocabs, tree_sizes, res_table, leaf_table, W_ih, b_ih, b_hh,
           W_iou, b_iou, U_iou, W_f, b_f, U_f, Wm, bm, Wv, bv):
    b_ihh = (b_ih + b_hh).reshape(1, 4 * H)
    hc_table = _tables_call(leaf_table, W_ih, b_ihh)

    hc_packed = lax.bitcast_convert_type(
        jnp.stack([hc_table[:, 0:H], hc_table[:, H:2 * H]], axis=-1), jnp.int32)
    ids_leaf = jnp.take(features, _LEAF_PERM).astype(jnp.int32)
    ids_int = jnp.take(features, _INT_PERM).astype(jnp.int32)
    hc = _sc_gather(hc_packed, ids_leaf)
    emb_int = _sc_gather(res_table, ids_int)

    consts = (U_iou.astype(jnp.bfloat16), U_f.astype(jnp.bfloat16),
              W_iou.astype(jnp.bfloat16), W_f.astype(jnp.bfloat16),
              b_iou.reshape(1, 3 * H), b_f.reshape(1, H))
    for l in (10, 9, 8, 7):
        cnt = T * 2 ** l
        blk = min(cnt, 1536)
        hc = _level_call(hc, hc, cnt // blk, emb_int, XW_BASES[l] // blk,
                         cnt, consts)

    eps = jax.random.normal(jax.random.key(42), (T, LATENT), jnp.float32)
    emb_small = lax.slice_in_dim(emb_int, 0, XW_BASES[7], axis=0)
    z, zm, zlv = _top_call(hc, emb_small, consts,
                           Wm, bm.reshape(1, LATENT),
                           Wv, bv.reshape(1, LATENT), eps)
    return (z, zm, zlv)
